# Initial kernel scaffold; baseline (speedup 1.0000x reference)
import jax
import jax.numpy as jnp
from jax.experimental import pallas as pl


def kernel(seq, embed, W1, b1, W2, b2, gamma, beta, Wg, bg, Wq, bq, Wo, bo):
    raise NotImplementedError("write your pallas kernel here")


# speedup vs baseline: 1.2589x; 1.2589x over previous
import functools

import jax
import jax.numpy as jnp
from jax.experimental import pallas as pl

HIDDEN_DIM = 64
VOCAB_SIZE = 64
MEMORY_SLOTS = 8
LP = 256
NEG = -1e30

INTERPRET = False


def _fused_kernel(L_real, BB, seq_ref, embed_ref, W1_ref, b1_ref, W2_ref,
                  b2_ref, gamma_ref, beta_ref, wg_ref, bg_ref, Wq_ref,
                  bq_ref, Wo_ref, bo_ref, out_ref):
    H = HIDDEN_DIM
    V = VOCAB_SIZE
    seq = seq_ref[...]

    oh = (seq[:, :, None] == jax.lax.broadcasted_iota(
        jnp.int32, (BB, LP, V), 2)).astype(jnp.float32)
    oh2 = oh.reshape(BB * LP, V)
    h0 = jnp.dot(oh2, embed_ref[...], preferred_element_type=jnp.float32, precision=jax.lax.Precision.HIGHEST)

    p1 = jnp.dot(h0, W1_ref[...], preferred_element_type=jnp.float32, precision=jax.lax.Precision.HIGHEST)
    ff1 = jnp.maximum(p1 + b1_ref[...], 0.0)
    ff = jnp.dot(ff1, W2_ref[...], preferred_element_type=jnp.float32, precision=jax.lax.Precision.HIGHEST)
    x = h0 + ff + b2_ref[...]
    mu = jnp.mean(x, axis=-1, keepdims=True)
    var = jnp.mean((x - mu) ** 2, axis=-1, keepdims=True)
    h2 = (x - mu) / jnp.sqrt(var + 1e-5) * gamma_ref[...] + beta_ref[...]
    h3 = h2.reshape(BB, LP, H)

    scores = jnp.sum(h3 * wg_ref[...].reshape(1, 1, H), axis=-1) + bg_ref[0, 0]
    pos = jax.lax.broadcasted_iota(jnp.int32, (BB, LP), 1)
    scores = jnp.where(pos < L_real, scores, NEG)

    mask = jnp.zeros((BB, LP), dtype=jnp.bool_)
    s = scores
    for _ in range(MEMORY_SLOTS):
        m = jnp.max(s, axis=1, keepdims=True)
        first = jnp.min(jnp.where(s == m, pos, LP), axis=1, keepdims=True)
        sel = pos == first
        mask = jnp.logical_or(mask, sel)
        s = jnp.where(sel, NEG, s)

    h_last = h3[:, L_real - 1, :]
    query = jnp.dot(h_last, Wq_ref[...], preferred_element_type=jnp.float32,
                    precision=jax.lax.Precision.HIGHEST) + bq_ref[...]
    logits = jnp.sum(h3 * query[:, None, :], axis=-1) * (H ** -0.5)
    logits = jnp.where(mask, logits, NEG)
    lmax = jnp.max(logits, axis=1, keepdims=True)
    e = jnp.where(mask, jnp.exp(logits - lmax), 0.0)
    attn = e / jnp.sum(e, axis=1, keepdims=True)
    ctx = jnp.sum(h3 * attn[:, :, None], axis=1)

    out_ref[...] = jnp.dot(ctx, Wo_ref[...],
                           preferred_element_type=jnp.float32, precision=jax.lax.Precision.HIGHEST) + bo_ref[...]


@jax.jit
def kernel(seq, embed, W1, b1, W2, b2, gamma, beta, Wg, bg, Wq, bq, Wo, bo):
    B, L = seq.shape
    H = HIDDEN_DIM
    V = VOCAB_SIZE
    BB = 32

    seq_p = jnp.pad(seq.astype(jnp.int32), ((0, 0), (0, LP - L)))
    row = lambda a: a.reshape(1, -1)
    full = lambda s: pl.BlockSpec(s, lambda i: (0, 0))

    grid = (B // BB,)
    out = pl.pallas_call(
        functools.partial(_fused_kernel, L, BB),
        grid=grid,
        in_specs=[
            pl.BlockSpec((BB, LP), lambda i: (i, 0)),
            full((V, H)),
            full((H, 2 * H)), full((1, 2 * H)),
            full((2 * H, H)), full((1, H)),
            full((1, H)), full((1, H)),
            full((1, H)), full((1, 1)),
            full((H, H)), full((1, H)),
            full((H, V)), full((1, V)),
        ],
        out_specs=pl.BlockSpec((BB, V), lambda i: (i, 0)),
        out_shape=jax.ShapeDtypeStruct((B, V), jnp.float32),
        interpret=INTERPRET,
    )(seq_p, embed, W1, row(b1), W2, row(b2), row(gamma), row(beta),
      Wg.reshape(1, H), bg.reshape(1, 1), Wq, row(bq), Wo, row(bo))
    return out


# speedup vs baseline: 2.4765x; 1.9672x over previous
import functools

import jax
import jax.numpy as jnp
from jax.experimental import pallas as pl

HIDDEN_DIM = 64
VOCAB_SIZE = 64
MEMORY_SLOTS = 8
LP = 256
NEG = -1e30

INTERPRET = False


def _fused_kernel(L_real, BB, seq_ref, embed_ref, W1_ref, b1_ref, W2_ref,
                  b2_ref, gamma_ref, beta_ref, wg_ref, bg_ref, Wq_ref,
                  bq_ref, Wo_ref, bo_ref, out_ref):
    H = HIDDEN_DIM
    V = VOCAB_SIZE
    seq = seq_ref[...]

    oh = (seq[:, :, None] == jax.lax.broadcasted_iota(
        jnp.int32, (BB, LP, V), 2)).astype(jnp.float32)
    oh2 = oh.reshape(BB * LP, V)
    h0 = jnp.dot(oh2, embed_ref[...], preferred_element_type=jnp.float32, precision=jax.lax.Precision.HIGHEST)

    p1 = jnp.dot(h0, W1_ref[...], preferred_element_type=jnp.float32)
    ff1 = jnp.maximum(p1 + b1_ref[...], 0.0)
    ff = jnp.dot(ff1, W2_ref[...], preferred_element_type=jnp.float32)
    x = h0 + ff + b2_ref[...]
    mu = jnp.mean(x, axis=-1, keepdims=True)
    var = jnp.mean((x - mu) ** 2, axis=-1, keepdims=True)
    h2 = (x - mu) / jnp.sqrt(var + 1e-5) * gamma_ref[...] + beta_ref[...]
    h3 = h2.reshape(BB, LP, H)

    scores = jnp.sum(h3 * wg_ref[...].reshape(1, 1, H), axis=-1) + bg_ref[0, 0]
    pos = jax.lax.broadcasted_iota(jnp.int32, (BB, LP), 1)
    scores = jnp.where(pos < L_real, scores, NEG)

    mask = jnp.zeros((BB, LP), dtype=jnp.bool_)
    s = scores
    for _ in range(MEMORY_SLOTS):
        m = jnp.max(s, axis=1, keepdims=True)
        first = jnp.min(jnp.where(s == m, pos, LP), axis=1, keepdims=True)
        sel = pos == first
        mask = jnp.logical_or(mask, sel)
        s = jnp.where(sel, NEG, s)

    h_last = h3[:, L_real - 1, :]
    query = jnp.dot(h_last, Wq_ref[...],
                    preferred_element_type=jnp.float32) + bq_ref[...]
    logits = jnp.sum(h3 * query[:, None, :], axis=-1) * (H ** -0.5)
    logits = jnp.where(mask, logits, NEG)
    lmax = jnp.max(logits, axis=1, keepdims=True)
    e = jnp.where(mask, jnp.exp(logits - lmax), 0.0)
    attn = e / jnp.sum(e, axis=1, keepdims=True)
    ctx = jnp.sum(h3 * attn[:, :, None], axis=1)

    out_ref[...] = jnp.dot(ctx, Wo_ref[...],
                           preferred_element_type=jnp.float32) + bo_ref[...]


@jax.jit
def kernel(seq, embed, W1, b1, W2, b2, gamma, beta, Wg, bg, Wq, bq, Wo, bo):
    B, L = seq.shape
    H = HIDDEN_DIM
    V = VOCAB_SIZE
    BB = 64

    seq_p = jnp.pad(seq.astype(jnp.int32), ((0, 0), (0, LP - L)))
    row = lambda a: a.reshape(1, -1)
    full = lambda s: pl.BlockSpec(s, lambda i: (0, 0))

    grid = (B // BB,)
    out = pl.pallas_call(
        functools.partial(_fused_kernel, L, BB),
        grid=grid,
        in_specs=[
            pl.BlockSpec((BB, LP), lambda i: (i, 0)),
            full((V, H)),
            full((H, 2 * H)), full((1, 2 * H)),
            full((2 * H, H)), full((1, H)),
            full((1, H)), full((1, H)),
            full((1, H)), full((1, 1)),
            full((H, H)), full((1, H)),
            full((H, V)), full((1, V)),
        ],
        out_specs=pl.BlockSpec((BB, V), lambda i: (i, 0)),
        out_shape=jax.ShapeDtypeStruct((B, V), jnp.float32),
        interpret=INTERPRET,
    )(seq_p, embed, W1, row(b1), W2, row(b2), row(gamma), row(beta),
      Wg.reshape(1, H), bg.reshape(1, 1), Wq, row(bq), Wo, row(bo))
    return out


# speedup vs baseline: 8.3680x; 3.3789x over previous
import functools

import jax
import jax.numpy as jnp
from jax.experimental import pallas as pl

HIDDEN_DIM = 64
VOCAB_SIZE = 64
MEMORY_SLOTS = 8
LP = 256
NEG = -1e30

INTERPRET = False


def _fused_kernel(L_real, BB, seq_ref, embed_ref, W1_ref, b1_ref, W2_ref,
                  b2_ref, gamma_ref, beta_ref, wg_ref, bg_ref, Wq_ref,
                  bq_ref, Wo_ref, bo_ref, out_ref):
    H = HIDDEN_DIM
    V = VOCAB_SIZE
    f32 = jnp.float32

    E = embed_ref[...]
    p1 = jnp.dot(E, W1_ref[...], preferred_element_type=f32)
    ff1 = jnp.maximum(p1 + b1_ref[...], 0.0)
    ff = jnp.dot(ff1, W2_ref[...], preferred_element_type=f32)
    x = E + ff + b2_ref[...]
    mu = jnp.mean(x, axis=-1, keepdims=True)
    var = jnp.mean((x - mu) ** 2, axis=-1, keepdims=True)
    HT = (x - mu) / jnp.sqrt(var + 1e-5) * gamma_ref[...] + beta_ref[...]

    st = jnp.sum(HT * wg_ref[...], axis=-1, keepdims=True) + bg_ref[0, 0]
    q_all = jnp.dot(HT, Wq_ref[...], preferred_element_type=f32) + bq_ref[...]
    LT = jax.lax.dot_general(HT, q_all, (((1,), (1,)), ((), ())),
                             preferred_element_type=f32) * (H ** -0.5)
    OT = jnp.dot(HT, Wo_ref[...], preferred_element_type=f32)
    G = (st > st.reshape(1, V)).astype(f32)

    seq = seq_ref[...]
    pos = jax.lax.broadcasted_iota(jnp.int32, (BB, LP), 1)
    tok = jax.lax.broadcasted_iota(jnp.int32, (BB, LP, V), 2)
    oh = jnp.where((seq[:, :, None] == tok) & (pos[:, :, None] < L_real),
                   1.0, 0.0)
    counts = jnp.sum(oh, axis=1)

    S = jnp.dot(counts, G, preferred_element_type=f32)
    taken = jnp.minimum(jnp.maximum(8.0 - S, 0.0), counts)

    q_tok = seq[:, L_real - 1][:, None]
    qoh = (q_tok == jax.lax.broadcasted_iota(
        jnp.int32, (BB, V), 1)).astype(f32)
    lg = jax.lax.dot_general(qoh, LT, (((1,), (1,)), ((), ())),
                             preferred_element_type=f32)
    sel = taken > 0.0
    lg_m = jnp.where(sel, lg, NEG)
    m = jnp.max(lg_m, axis=1, keepdims=True)
    e = jnp.where(sel, taken * jnp.exp(lg - m), 0.0)
    w = e / jnp.sum(e, axis=1, keepdims=True)
    out_ref[...] = jnp.dot(w, OT, preferred_element_type=f32) + bo_ref[...]


@jax.jit
def kernel(seq, embed, W1, b1, W2, b2, gamma, beta, Wg, bg, Wq, bq, Wo, bo):
    B, L = seq.shape
    H = HIDDEN_DIM
    V = VOCAB_SIZE
    BB = 256

    seq_p = jnp.pad(seq.astype(jnp.int32), ((0, 0), (0, LP - L)))
    row = lambda a: a.reshape(1, -1)
    full = lambda s: pl.BlockSpec(s, lambda i: (0, 0))

    grid = (B // BB,)
    out = pl.pallas_call(
        functools.partial(_fused_kernel, L, BB),
        grid=grid,
        in_specs=[
            pl.BlockSpec((BB, LP), lambda i: (i, 0)),
            full((V, H)),
            full((H, 2 * H)), full((1, 2 * H)),
            full((2 * H, H)), full((1, H)),
            full((1, H)), full((1, H)),
            full((1, H)), full((1, 1)),
            full((H, H)), full((1, H)),
            full((H, V)), full((1, V)),
        ],
        out_specs=pl.BlockSpec((BB, V), lambda i: (i, 0)),
        out_shape=jax.ShapeDtypeStruct((B, V), jnp.float32),
        interpret=INTERPRET,
    )(seq_p, embed, W1, row(b1), W2, row(b2), row(gamma), row(beta),
      Wg.reshape(1, H), bg.reshape(1, 1), Wq, row(bq), Wo, row(bo))
    return out


# speedup vs baseline: 61.8175x; 7.3874x over previous
import functools

import jax
import jax.numpy as jnp
from jax.experimental import pallas as pl

HIDDEN_DIM = 64
VOCAB_SIZE = 64
MEMORY_SLOTS = 8
LP = 256
NEG = -1e30

INTERPRET = False


def _fused_kernel(L_real, BB, seq_ref, embed_ref, W1_ref, b1_ref, W2_ref,
                  b2_ref, gamma_ref, beta_ref, wg_ref, bg_ref, Wq_ref,
                  bq_ref, Wo_ref, bo_ref, out_ref):
    H = HIDDEN_DIM
    V = VOCAB_SIZE
    f32 = jnp.float32

    E = embed_ref[...]
    p1 = jnp.dot(E, W1_ref[...], preferred_element_type=f32)
    ff1 = jnp.maximum(p1 + b1_ref[...], 0.0)
    ff = jnp.dot(ff1, W2_ref[...], preferred_element_type=f32)
    x = E + ff + b2_ref[...]
    mu = jnp.mean(x, axis=-1, keepdims=True)
    var = jnp.mean((x - mu) ** 2, axis=-1, keepdims=True)
    HT = (x - mu) / jnp.sqrt(var + 1e-5) * gamma_ref[...] + beta_ref[...]

    st = jnp.sum(HT * wg_ref[...], axis=-1, keepdims=True) + bg_ref[0, 0]
    q_all = jnp.dot(HT, Wq_ref[...], preferred_element_type=f32) + bq_ref[...]
    LT = jax.lax.dot_general(HT, q_all, (((1,), (1,)), ((), ())),
                             preferred_element_type=f32) * (H ** -0.5)
    OT = jnp.dot(HT, Wo_ref[...], preferred_element_type=f32)
    G = (st > st.reshape(1, V)).astype(f32)

    seq = seq_ref[...]
    pos = jax.lax.broadcasted_iota(jnp.int32, (BB, LP), 1)
    seqv = jnp.where(pos < L_real, seq, -1)
    seq_t = seqv.T
    tok = jax.lax.broadcasted_iota(jnp.int32, (V, LP, BB), 0)
    oh = jnp.where(seq_t[None, :, :] == tok, 1.0, 0.0)
    counts = jnp.sum(oh, axis=1).T

    S = jnp.dot(counts, G, preferred_element_type=f32)
    taken = jnp.minimum(jnp.maximum(8.0 - S, 0.0), counts)

    q_tok = seq[:, L_real - 1][:, None]
    qoh = (q_tok == jax.lax.broadcasted_iota(
        jnp.int32, (BB, V), 1)).astype(f32)
    lg = jax.lax.dot_general(qoh, LT, (((1,), (1,)), ((), ())),
                             preferred_element_type=f32)
    sel = taken > 0.0
    lg_m = jnp.where(sel, lg, NEG)
    m = jnp.max(lg_m, axis=1, keepdims=True)
    e = jnp.where(sel, taken * jnp.exp(lg - m), 0.0)
    w = e / jnp.sum(e, axis=1, keepdims=True)
    out_ref[...] = jnp.dot(w, OT, preferred_element_type=f32) + bo_ref[...]


@jax.jit
def kernel(seq, embed, W1, b1, W2, b2, gamma, beta, Wg, bg, Wq, bq, Wo, bo):
    B, L = seq.shape
    H = HIDDEN_DIM
    V = VOCAB_SIZE
    BB = 256

    seq_p = jnp.pad(seq.astype(jnp.int32), ((0, 0), (0, LP - L)))
    row = lambda a: a.reshape(1, -1)
    full = lambda s: pl.BlockSpec(s, lambda i: (0, 0))

    grid = (B // BB,)
    out = pl.pallas_call(
        functools.partial(_fused_kernel, L, BB),
        grid=grid,
        in_specs=[
            pl.BlockSpec((BB, LP), lambda i: (i, 0)),
            full((V, H)),
            full((H, 2 * H)), full((1, 2 * H)),
            full((2 * H, H)), full((1, H)),
            full((1, H)), full((1, H)),
            full((1, H)), full((1, 1)),
            full((H, H)), full((1, H)),
            full((H, V)), full((1, V)),
        ],
        out_specs=pl.BlockSpec((BB, V), lambda i: (i, 0)),
        out_shape=jax.ShapeDtypeStruct((B, V), jnp.float32),
        interpret=INTERPRET,
    )(seq_p, embed, W1, row(b1), W2, row(b2), row(gamma), row(beta),
      Wg.reshape(1, H), bg.reshape(1, 1), Wq, row(bq), Wo, row(bo))
    return out


# speedup vs baseline: 92.8325x; 1.5017x over previous
import functools

import jax
import jax.numpy as jnp
from jax.experimental import pallas as pl

HIDDEN_DIM = 64
VOCAB_SIZE = 64
MEMORY_SLOTS = 8
NEG = -1e30

INTERPRET = False


def _fused_kernel(L_real, BB, seq_ref, embed_ref, W1_ref, b1_ref, W2_ref,
                  b2_ref, gamma_ref, beta_ref, wg_ref, bg_ref, Wq_ref,
                  bq_ref, Wo_ref, bo_ref, out_ref):
    H = HIDDEN_DIM
    V = VOCAB_SIZE
    f32 = jnp.float32

    E = embed_ref[...]
    p1 = jnp.dot(E, W1_ref[...], preferred_element_type=f32)
    ff1 = jnp.maximum(p1 + b1_ref[...], 0.0)
    ff = jnp.dot(ff1, W2_ref[...], preferred_element_type=f32)
    x = E + ff + b2_ref[...]
    mu = jnp.mean(x, axis=-1, keepdims=True)
    var = jnp.mean((x - mu) ** 2, axis=-1, keepdims=True)
    HT = (x - mu) / jnp.sqrt(var + 1e-5) * gamma_ref[...] + beta_ref[...]

    st = jnp.sum(HT * wg_ref[...], axis=-1, keepdims=True) + bg_ref[0, 0]
    q_all = jnp.dot(HT, Wq_ref[...], preferred_element_type=f32) + bq_ref[...]
    LT = jax.lax.dot_general(HT, q_all, (((1,), (1,)), ((), ())),
                             preferred_element_type=f32) * (H ** -0.5)
    OT = jnp.dot(HT, Wo_ref[...], preferred_element_type=f32)
    G = (st > st.reshape(1, V)).astype(f32)

    seq = seq_ref[...]
    seq_t = seq.T
    tok = jax.lax.broadcasted_iota(jnp.int32, (V, L_real, BB), 0)
    oh = jnp.where(seq_t[None, :, :] == tok, 1.0, 0.0)
    counts = jnp.sum(oh, axis=1).T

    S = jnp.dot(counts, G, preferred_element_type=f32)
    taken = jnp.minimum(jnp.maximum(8.0 - S, 0.0), counts)

    q_tok = seq[:, L_real - 1][:, None]
    qoh = (q_tok == jax.lax.broadcasted_iota(
        jnp.int32, (BB, V), 1)).astype(f32)
    lg = jax.lax.dot_general(qoh, LT, (((1,), (1,)), ((), ())),
                             preferred_element_type=f32)
    sel = taken > 0.0
    lg_m = jnp.where(sel, lg, NEG)
    m = jnp.max(lg_m, axis=1, keepdims=True)
    e = jnp.where(sel, taken * jnp.exp(lg - m), 0.0)
    w = e / jnp.sum(e, axis=1, keepdims=True)
    out_ref[...] = jnp.dot(w, OT, preferred_element_type=f32) + bo_ref[...]


@jax.jit
def kernel(seq, embed, W1, b1, W2, b2, gamma, beta, Wg, bg, Wq, bq, Wo, bo):
    B, L = seq.shape
    H = HIDDEN_DIM
    V = VOCAB_SIZE
    BB = 1024

    seq_p = seq.astype(jnp.int32)
    row = lambda a: a.reshape(1, -1)
    full = lambda s: pl.BlockSpec(s, lambda i: (0, 0))

    grid = (B // BB,)
    out = pl.pallas_call(
        functools.partial(_fused_kernel, L, BB),
        grid=grid,
        in_specs=[
            pl.BlockSpec((BB, L), lambda i: (i, 0)),
            full((V, H)),
            full((H, 2 * H)), full((1, 2 * H)),
            full((2 * H, H)), full((1, H)),
            full((1, H)), full((1, H)),
            full((1, H)), full((1, 1)),
            full((H, H)), full((1, H)),
            full((H, V)), full((1, V)),
        ],
        out_specs=pl.BlockSpec((BB, V), lambda i: (i, 0)),
        out_shape=jax.ShapeDtypeStruct((B, V), jnp.float32),
        interpret=INTERPRET,
    )(seq_p, embed, W1, row(b1), W2, row(b2), row(gamma), row(beta),
      Wg.reshape(1, H), bg.reshape(1, 1), Wq, row(bq), Wo, row(bo))
    return out


# speedup vs baseline: 114.3139x; 1.2314x over previous
import functools

import jax
import jax.numpy as jnp
from jax.experimental import pallas as pl

HIDDEN_DIM = 64
VOCAB_SIZE = 64
MEMORY_SLOTS = 8
NEG = -1e30

INTERPRET = False


def _fused_kernel(L_real, BB, seq_ref, embed_ref, W1_ref, b1_ref, W2_ref,
                  b2_ref, gamma_ref, beta_ref, wg_ref, bg_ref, Wq_ref,
                  bq_ref, Wo_ref, bo_ref, out_ref):
    H = HIDDEN_DIM
    V = VOCAB_SIZE
    f32 = jnp.float32

    E = embed_ref[...]
    p1 = jnp.dot(E, W1_ref[...], preferred_element_type=f32)
    ff1 = jnp.maximum(p1 + b1_ref[...], 0.0)
    ff = jnp.dot(ff1, W2_ref[...], preferred_element_type=f32)
    x = E + ff + b2_ref[...]
    mu = jnp.mean(x, axis=-1, keepdims=True)
    var = jnp.mean((x - mu) ** 2, axis=-1, keepdims=True)
    HT = (x - mu) / jnp.sqrt(var + 1e-5) * gamma_ref[...] + beta_ref[...]

    st = jnp.sum(HT * wg_ref[...], axis=-1, keepdims=True) + bg_ref[0, 0]
    q_all = jnp.dot(HT, Wq_ref[...], preferred_element_type=f32) + bq_ref[...]
    LT = jax.lax.dot_general(HT, q_all, (((1,), (1,)), ((), ())),
                             preferred_element_type=f32) * (H ** -0.5)
    OT = jnp.dot(HT, Wo_ref[...], preferred_element_type=f32)
    G = (st > st.reshape(1, V)).astype(f32)

    seq = seq_ref[...]
    seq_t = seq.T
    tok = jax.lax.broadcasted_iota(jnp.int32, (V, L_real, BB), 0)
    oh = jnp.where(seq_t[None, :, :] == tok, 1.0, 0.0)
    ones_l = jnp.ones((V, 1, L_real), dtype=f32)
    counts = jax.lax.dot_general(
        ones_l, oh, (((2,), (1,)), ((0,), (0,))),
        preferred_element_type=f32).reshape(V, BB).T

    S = jnp.dot(counts, G, preferred_element_type=f32)
    taken = jnp.minimum(jnp.maximum(8.0 - S, 0.0), counts)

    q_tok = seq[:, L_real - 1][:, None]
    qoh = (q_tok == jax.lax.broadcasted_iota(
        jnp.int32, (BB, V), 1)).astype(f32)
    lg = jax.lax.dot_general(qoh, LT, (((1,), (1,)), ((), ())),
                             preferred_element_type=f32)
    sel = taken > 0.0
    lg_m = jnp.where(sel, lg, NEG)
    m = jnp.max(lg_m, axis=1, keepdims=True)
    e = jnp.where(sel, taken * jnp.exp(lg - m), 0.0)
    w = e / jnp.sum(e, axis=1, keepdims=True)
    out_ref[...] = jnp.dot(w, OT, preferred_element_type=f32) + bo_ref[...]


@jax.jit
def kernel(seq, embed, W1, b1, W2, b2, gamma, beta, Wg, bg, Wq, bq, Wo, bo):
    B, L = seq.shape
    H = HIDDEN_DIM
    V = VOCAB_SIZE
    BB = 1024

    seq_p = seq.astype(jnp.int32)
    row = lambda a: a.reshape(1, -1)
    full = lambda s: pl.BlockSpec(s, lambda i: (0, 0))

    grid = (B // BB,)
    out = pl.pallas_call(
        functools.partial(_fused_kernel, L, BB),
        grid=grid,
        in_specs=[
            pl.BlockSpec((BB, L), lambda i: (i, 0)),
            full((V, H)),
            full((H, 2 * H)), full((1, 2 * H)),
            full((2 * H, H)), full((1, H)),
            full((1, H)), full((1, H)),
            full((1, H)), full((1, 1)),
            full((H, H)), full((1, H)),
            full((H, V)), full((1, V)),
        ],
        out_specs=pl.BlockSpec((BB, V), lambda i: (i, 0)),
        out_shape=jax.ShapeDtypeStruct((B, V), jnp.float32),
        interpret=INTERPRET,
    )(seq_p, embed, W1, row(b1), W2, row(b2), row(gamma), row(beta),
      Wg.reshape(1, H), bg.reshape(1, 1), Wq, row(bq), Wo, row(bo))
    return out
